# baseline (speedup 1.0000x reference)
import jax
import jax.numpy as jnp
from jax import lax
from jax.experimental import pallas as pl
from jax.experimental.pallas import tpu as pltpu
from jax.experimental.pallas import tpu_sc as plsc

N_GLOBAL = 100000
N2 = 10000
E = 320000
D = 128

NC = 2
NS = 16
NW = NC * NS
B = 128
RPT = 160
DRPT = 80
E_PAD = NS * RPT * B
N2P = 10240
HALF = N2P // NC
ACC_H = HALF + 8
ACC_ROWS = N2P + 16
STRIPE = HALF // NS
DSTRIPE = N2P // NS

G_CHUNK = 512
G_TILES = N2P // G_CHUNK

_MESH = plsc.VectorSubcoreMesh(core_axis_name="c", subcore_axis_name="s")


def _fill_rows(ref, nrows, value):
    ncol16 = ref.shape[1] // 16

    def body(i, _):
        for j in range(ncol16):
            ref[i, pl.ds(j * 16, 16)] = jnp.full((16,), value, jnp.float32)
        return 0

    lax.fori_loop(0, nrows, body, 0)


def _agg_body(h_hbm, srcm, dstm, out_hbm,
              sidx, didx, rows_a, zbuf, acc, gsem):
    c = lax.axis_index("c")
    s = lax.axis_index("s")
    base = c * HALF

    _fill_rows(zbuf, 80, 0.0)
    for q in range(4):
        pltpu.sync_copy(zbuf, acc.at[pl.ds(s * STRIPE + q * 80, 80)])
    plsc.subcore_barrier()

    def grp(g, _):
        gbase = s * RPT + g * 4
        pltpu.sync_copy(srcm.at[pl.ds(gbase, 4)], sidx)
        pltpu.sync_copy(dstm.at[pl.ds(gbase, 4)], didx)
        for r in range(4):
            for kk in range(8):
                v = didx[r, pl.ds(kk * 16, 16)] - base
                keep = (v >= 0) & (v < HALF)
                didx[r, pl.ds(kk * 16, 16)] = jnp.where(keep, v, HALF)
        for q in range(4):
            pltpu.async_copy(h_hbm.at[sidx.at[q]], rows_a, gsem).wait()
            pltpu.sync_copy(rows_a, acc.at[didx.at[q]], add=True)
        return 0

    lax.fori_loop(0, RPT // 4, grp, 0)

    plsc.subcore_barrier()
    pltpu.sync_copy(acc.at[pl.ds(s * STRIPE, STRIPE)],
                    out_hbm.at[pl.ds(c * HALF + s * STRIPE, STRIPE)])


_agg_call = pl.kernel(
    _agg_body,
    out_type=jax.ShapeDtypeStruct((N2P, D), jnp.float32),
    mesh=_MESH,
    scratch_types=[
        pltpu.VMEM((4, B), jnp.int32),
        pltpu.VMEM((4, B), jnp.int32),
        pltpu.VMEM((B, D), jnp.float32),
        pltpu.VMEM((80, D), jnp.float32),
        pltpu.VMEM_SHARED((ACC_H, D), jnp.float32),
        pltpu.SemaphoreType.DMA,
    ],
)


def _prep_body(dstm, g2m, gh1_hbm, gh2_hbm,
               deg_out, gath1, gath2,
               didx, g2v, ones_v, rows, zbuf, dacc, sem):
    c = lax.axis_index("c")
    s = lax.axis_index("s")
    wid = c * NS + s
    base = c * HALF

    _fill_rows(ones_v, B, 1.0)
    _fill_rows(zbuf, 80, 0.0)
    for q in range(4):
        pltpu.sync_copy(zbuf, dacc.at[pl.ds(s * STRIPE + q * 80, 80)])
    plsc.subcore_barrier()

    def grp(g, _):
        pltpu.sync_copy(dstm.at[pl.ds(s * RPT + g * 4, 4)], didx)
        for r in range(4):
            for kk in range(8):
                v = didx[r, pl.ds(kk * 16, 16)] - base
                keep = (v >= 0) & (v < HALF)
                didx[r, pl.ds(kk * 16, 16)] = jnp.where(keep, v, HALF)
        for q in range(4):
            pltpu.sync_copy(ones_v, dacc.at[didx.at[q]], add=True)
        return 0

    lax.fori_loop(0, RPT // 4, grp, 0)

    @pl.when(wid < G_TILES)
    def _():
        pltpu.sync_copy(g2m.at[pl.ds(wid * 4, 4)], g2v)
        for q in range(4):
            pltpu.async_copy(gh1_hbm.at[g2v.at[q]], rows, sem).wait()
            pltpu.sync_copy(
                rows, gath1.at[pl.ds(wid * G_CHUNK + q * B, B)])
            pltpu.async_copy(gh2_hbm.at[g2v.at[q]], rows, sem).wait()
            pltpu.sync_copy(
                rows, gath2.at[pl.ds(wid * G_CHUNK + q * B, B)])

    plsc.subcore_barrier()
    pltpu.sync_copy(dacc.at[pl.ds(s * STRIPE, STRIPE)],
                    deg_out.at[pl.ds(c * HALF + s * STRIPE, STRIPE)])


_prep_call = pl.kernel(
    _prep_body,
    out_type=[
        jax.ShapeDtypeStruct((N2P, D), jnp.float32),
        jax.ShapeDtypeStruct((N2P, D), jnp.float32),
        jax.ShapeDtypeStruct((N2P, D), jnp.float32),
    ],
    mesh=_MESH,
    scratch_types=[
        pltpu.VMEM((4, B), jnp.int32),
        pltpu.VMEM((4, B), jnp.int32),
        pltpu.VMEM((B, D), jnp.float32),
        pltpu.VMEM((B, D), jnp.float32),
        pltpu.VMEM((80, D), jnp.float32),
        pltpu.VMEM_SHARED((ACC_H, D), jnp.float32),
        pltpu.SemaphoreType.DMA,
    ],
)


RB = 1000


def _layer_body(h_ref, agg_ref, degp_ref, ws_ref, wn_ref, b_ref,
                mask_ref, gath_ref, hm_ref, hr_ref):
    deg = degp_ref[:, 0:1]
    hn = agg_ref[...] / jnp.maximum(deg, 1.0)
    out = (
        jnp.dot(h_ref[...], ws_ref[...], preferred_element_type=jnp.float32,
                precision=lax.Precision.HIGHEST)
        + jnp.dot(hn, wn_ref[...], preferred_element_type=jnp.float32,
                  precision=lax.Precision.HIGHEST)
        + b_ref[...]
    )
    hm = jnp.where(mask_ref[...] != 0, out, gath_ref[...])
    hm_ref[...] = hm
    hr_ref[...] = jnp.maximum(hm, 0.0)


def _layer_call(h, agg, degp, ws, wn, b, mask, gath):
    return pl.pallas_call(
        _layer_body,
        grid=(N2 // RB,),
        in_specs=[
            pl.BlockSpec((RB, D), lambda i: (i, 0)),
            pl.BlockSpec((RB, D), lambda i: (i, 0)),
            pl.BlockSpec((RB, D), lambda i: (i, 0)),
            pl.BlockSpec((D, D), lambda i: (0, 0)),
            pl.BlockSpec((D, D), lambda i: (0, 0)),
            pl.BlockSpec((1, D), lambda i: (0, 0)),
            pl.BlockSpec((RB, 1), lambda i: (i, 0)),
            pl.BlockSpec((RB, D), lambda i: (i, 0)),
        ],
        out_specs=[
            pl.BlockSpec((RB, D), lambda i: (i, 0)),
            pl.BlockSpec((RB, D), lambda i: (i, 0)),
        ],
        out_shape=[
            jax.ShapeDtypeStruct((N2, D), jnp.float32),
            jax.ShapeDtypeStruct((N2, D), jnp.float32),
        ],
    )(h, agg, degp, ws, wn, b, mask, gath)


def _layer3_body(h_ref, agg_ref, degp_ref, ws_ref, wn_ref, b_ref, out_ref):
    deg = degp_ref[:, 0:1]
    hn = agg_ref[...] / jnp.maximum(deg, 1.0)
    out_ref[...] = (
        jnp.dot(h_ref[...], ws_ref[...], preferred_element_type=jnp.float32,
                precision=lax.Precision.HIGHEST)
        + jnp.dot(hn, wn_ref[...], preferred_element_type=jnp.float32,
                  precision=lax.Precision.HIGHEST)
        + b_ref[...]
    )


def _layer3_call(h, agg, degp, ws, wn, b):
    return pl.pallas_call(
        _layer3_body,
        grid=(N2 // RB,),
        in_specs=[
            pl.BlockSpec((RB, D), lambda i: (i, 0)),
            pl.BlockSpec((RB, D), lambda i: (i, 0)),
            pl.BlockSpec((RB, D), lambda i: (i, 0)),
            pl.BlockSpec((D, D), lambda i: (0, 0)),
            pl.BlockSpec((D, D), lambda i: (0, 0)),
            pl.BlockSpec((1, D), lambda i: (0, 0)),
        ],
        out_specs=[pl.BlockSpec((RB, D), lambda i: (i, 0))],
        out_shape=[jax.ShapeDtypeStruct((N2, D), jnp.float32)],
    )(h, agg, degp, ws, wn, b)[0]


CB = 2000


def _copy_body(a_ref, b_ref, oa_ref, ob_ref):
    oa_ref[...] = a_ref[...]
    ob_ref[...] = b_ref[...]


def _copy_tables(t1, t2):
    return pl.pallas_call(
        _copy_body,
        grid=(N_GLOBAL // CB,),
        in_specs=[
            pl.BlockSpec((CB, D), lambda i: (i, 0)),
            pl.BlockSpec((CB, D), lambda i: (i, 0)),
        ],
        out_specs=[
            pl.BlockSpec((CB, D), lambda i: (i, 0)),
            pl.BlockSpec((CB, D), lambda i: (i, 0)),
        ],
        out_shape=[
            jax.ShapeDtypeStruct((N_GLOBAL, D), jnp.float32),
            jax.ShapeDtypeStruct((N_GLOBAL, D), jnp.float32),
        ],
    )(t1, t2)


def _scatter_body(ids_ref, v1_ref, v2_ref, t1_ref, t2_ref, o1_ref, o2_ref):
    del ids_ref, t1_ref, t2_ref
    o1_ref[...] = v1_ref[...]
    o2_ref[...] = v2_ref[...]


def _scatter_tables(ids, v1, v2, t1, t2):
    grid_spec = pltpu.PrefetchScalarGridSpec(
        num_scalar_prefetch=1,
        grid=(N2,),
        in_specs=[
            pl.BlockSpec((1, 1, D), lambda i, ids: (i, 0, 0)),
            pl.BlockSpec((1, 1, D), lambda i, ids: (i, 0, 0)),
            pl.BlockSpec(memory_space=pl.ANY),
            pl.BlockSpec(memory_space=pl.ANY),
        ],
        out_specs=[
            pl.BlockSpec((1, 1, D), lambda i, ids: (ids[i], 0, 0)),
            pl.BlockSpec((1, 1, D), lambda i, ids: (ids[i], 0, 0)),
        ],
    )
    o1, o2 = pl.pallas_call(
        _scatter_body,
        grid_spec=grid_spec,
        out_shape=[
            jax.ShapeDtypeStruct((N_GLOBAL, 1, D), jnp.float32),
            jax.ShapeDtypeStruct((N_GLOBAL, 1, D), jnp.float32),
        ],
        input_output_aliases={3: 0, 4: 1},
    )(ids, v1.reshape(N2, 1, D), v2.reshape(N2, 1, D),
      t1.reshape(N_GLOBAL, 1, D), t2.reshape(N_GLOBAL, 1, D))
    return o1.reshape(N_GLOBAL, D), o2.reshape(N_GLOBAL, D)


def kernel(g2_feat, edge_index, g1_ids, g2_ids, gh_feat, gh2_feat,
           W1_self, W1_neigh, b1, W2_self, W2_neigh, b2,
           W3_self, W3_neigh, b3):
    src = edge_index[0]
    dst = edge_index[1]

    pad = E_PAD - E
    srcm = jnp.concatenate([src, jnp.zeros((pad,), jnp.int32)]).reshape(-1, B)
    dstm = jnp.concatenate([dst, jnp.full((pad,), N2P, jnp.int32)]).reshape(-1, B)

    memb = jnp.zeros((N_GLOBAL,), jnp.int32).at[g1_ids].set(1)
    mask = memb[g2_ids].astype(jnp.float32)[:, None]

    b1r = b1.reshape(1, D)
    b2r = b2.reshape(1, D)
    w3s = jnp.pad(W3_self, ((0, 0), (0, D - W3_self.shape[1])))
    w3n = jnp.pad(W3_neigh, ((0, 0), (0, D - W3_neigh.shape[1])))
    b3r = jnp.pad(b3, (0, D - b3.shape[0])).reshape(1, D)

    g2pad = jnp.concatenate(
        [g2_ids, jnp.zeros((N2P - N2,), jnp.int32)]).reshape(-1, B)
    degp, gath1, gath2 = _prep_call(dstm, g2pad, gh_feat, gh2_feat)
    gath1 = gath1[:N2]
    gath2 = gath2[:N2]

    agg1 = _agg_call(g2_feat, srcm, dstm)[:N2]
    h1m, h1r = _layer_call(g2_feat, agg1, degp, W1_self, W1_neigh, b1r,
                           mask, gath1)

    agg2 = _agg_call(h1r, srcm, dstm)[:N2]
    h2m, h2r = _layer_call(h1r, agg2, degp, W2_self, W2_neigh, b2r,
                           mask, gath2)

    agg3 = _agg_call(h2r, srcm, dstm)[:N2]
    h3 = _layer3_call(h2r, agg3, degp, w3s, w3n, b3r)[:, :W3_self.shape[1]]

    t1, t2 = _copy_tables(gh_feat, gh2_feat)
    gh_new, gh2_new = _scatter_tables(g2_ids, h1m, h2m, t1, t2)

    return h3, gh_new, gh2_new


# speedup vs baseline: 1.0080x; 1.0080x over previous
import jax
import jax.numpy as jnp
from jax import lax
from jax.experimental import pallas as pl
from jax.experimental.pallas import tpu as pltpu
from jax.experimental.pallas import tpu_sc as plsc

N_GLOBAL = 100000
N2 = 10000
E = 320000
D = 128

NC = 2
NS = 16
NW = NC * NS
B = 128
RPT = 160
DRPT = 80
E_PAD = NS * RPT * B
N2P = 10240
HALF = N2P // NC
ACC_H = HALF + 8
ACC_ROWS = N2P + 16
STRIPE = HALF // NS
DSTRIPE = N2P // NS

G_CHUNK = 512
G_TILES = N2P // G_CHUNK

_MESH = plsc.VectorSubcoreMesh(core_axis_name="c", subcore_axis_name="s")


def _fill_rows(ref, nrows, value):
    ncol16 = ref.shape[1] // 16

    def body(i, _):
        for j in range(ncol16):
            ref[i, pl.ds(j * 16, 16)] = jnp.full((16,), value, jnp.float32)
        return 0

    lax.fori_loop(0, nrows, body, 0)


def _agg_body(h_hbm, srcm, dstm, out_hbm,
              sidx, didx, rows_a, rows_b, zbuf, acc, gsem, gsem2):
    c = lax.axis_index("c")
    s = lax.axis_index("s")
    base = c * HALF

    _fill_rows(zbuf, 80, 0.0)
    for q in range(4):
        pltpu.sync_copy(zbuf, acc.at[pl.ds(s * STRIPE + q * 80, 80)])
    plsc.subcore_barrier()

    def grp(g, _):
        gbase = s * RPT + g * 4
        pltpu.sync_copy(srcm.at[pl.ds(gbase, 4)], sidx)
        pltpu.sync_copy(dstm.at[pl.ds(gbase, 4)], didx)
        for r in range(4):
            for kk in range(8):
                v = didx[r, pl.ds(kk * 16, 16)] - base
                keep = (v >= 0) & (v < HALF)
                didx[r, pl.ds(kk * 16, 16)] = jnp.where(keep, v, HALF)
        pltpu.async_copy(h_hbm.at[sidx.at[0]], rows_a, gsem)
        pltpu.async_copy(h_hbm.at[sidx.at[1]], rows_b, gsem2)
        pltpu.make_async_copy(h_hbm.at[sidx.at[0]], rows_a, gsem).wait()
        pltpu.sync_copy(rows_a, acc.at[didx.at[0]], add=True)
        pltpu.async_copy(h_hbm.at[sidx.at[2]], rows_a, gsem)
        pltpu.make_async_copy(h_hbm.at[sidx.at[1]], rows_b, gsem2).wait()
        pltpu.sync_copy(rows_b, acc.at[didx.at[1]], add=True)
        pltpu.async_copy(h_hbm.at[sidx.at[3]], rows_b, gsem2)
        pltpu.make_async_copy(h_hbm.at[sidx.at[2]], rows_a, gsem).wait()
        pltpu.sync_copy(rows_a, acc.at[didx.at[2]], add=True)
        pltpu.make_async_copy(h_hbm.at[sidx.at[3]], rows_b, gsem2).wait()
        pltpu.sync_copy(rows_b, acc.at[didx.at[3]], add=True)
        return 0

    lax.fori_loop(0, RPT // 4, grp, 0)

    plsc.subcore_barrier()
    pltpu.sync_copy(acc.at[pl.ds(s * STRIPE, STRIPE)],
                    out_hbm.at[pl.ds(c * HALF + s * STRIPE, STRIPE)])


_agg_call = pl.kernel(
    _agg_body,
    out_type=jax.ShapeDtypeStruct((N2P, D), jnp.float32),
    mesh=_MESH,
    scratch_types=[
        pltpu.VMEM((4, B), jnp.int32),
        pltpu.VMEM((4, B), jnp.int32),
        pltpu.VMEM((B, D), jnp.float32),
        pltpu.VMEM((B, D), jnp.float32),
        pltpu.VMEM((80, D), jnp.float32),
        pltpu.VMEM_SHARED((ACC_H, D), jnp.float32),
        pltpu.SemaphoreType.DMA,
        pltpu.SemaphoreType.DMA,
    ],
)


def _prep_body(dstm, g2m, gh1_hbm, gh2_hbm,
               deg_out, gath1, gath2,
               didx, g2v, ones_v, rows, zbuf, dacc, sem):
    c = lax.axis_index("c")
    s = lax.axis_index("s")
    wid = c * NS + s
    base = c * HALF

    _fill_rows(ones_v, B, 1.0)
    _fill_rows(zbuf, 80, 0.0)
    for q in range(4):
        pltpu.sync_copy(zbuf, dacc.at[pl.ds(s * STRIPE + q * 80, 80)])
    plsc.subcore_barrier()

    def grp(g, _):
        pltpu.sync_copy(dstm.at[pl.ds(s * RPT + g * 4, 4)], didx)
        for r in range(4):
            for kk in range(8):
                v = didx[r, pl.ds(kk * 16, 16)] - base
                keep = (v >= 0) & (v < HALF)
                didx[r, pl.ds(kk * 16, 16)] = jnp.where(keep, v, HALF)
        for q in range(4):
            pltpu.sync_copy(ones_v, dacc.at[didx.at[q]], add=True)
        return 0

    lax.fori_loop(0, RPT // 4, grp, 0)

    @pl.when(wid < G_TILES)
    def _():
        pltpu.sync_copy(g2m.at[pl.ds(wid * 4, 4)], g2v)
        for q in range(4):
            pltpu.async_copy(gh1_hbm.at[g2v.at[q]], rows, sem).wait()
            pltpu.sync_copy(
                rows, gath1.at[pl.ds(wid * G_CHUNK + q * B, B)])
            pltpu.async_copy(gh2_hbm.at[g2v.at[q]], rows, sem).wait()
            pltpu.sync_copy(
                rows, gath2.at[pl.ds(wid * G_CHUNK + q * B, B)])

    plsc.subcore_barrier()
    pltpu.sync_copy(dacc.at[pl.ds(s * STRIPE, STRIPE)],
                    deg_out.at[pl.ds(c * HALF + s * STRIPE, STRIPE)])


_prep_call = pl.kernel(
    _prep_body,
    out_type=[
        jax.ShapeDtypeStruct((N2P, D), jnp.float32),
        jax.ShapeDtypeStruct((N2P, D), jnp.float32),
        jax.ShapeDtypeStruct((N2P, D), jnp.float32),
    ],
    mesh=_MESH,
    scratch_types=[
        pltpu.VMEM((4, B), jnp.int32),
        pltpu.VMEM((4, B), jnp.int32),
        pltpu.VMEM((B, D), jnp.float32),
        pltpu.VMEM((B, D), jnp.float32),
        pltpu.VMEM((80, D), jnp.float32),
        pltpu.VMEM_SHARED((ACC_H, D), jnp.float32),
        pltpu.SemaphoreType.DMA,
    ],
)


RB = 1000


def _layer_body(h_ref, agg_ref, degp_ref, ws_ref, wn_ref, b_ref,
                mask_ref, gath_ref, hm_ref, hr_ref):
    deg = degp_ref[:, 0:1]
    hn = agg_ref[...] / jnp.maximum(deg, 1.0)
    out = (
        jnp.dot(h_ref[...], ws_ref[...], preferred_element_type=jnp.float32,
                precision=lax.Precision.HIGHEST)
        + jnp.dot(hn, wn_ref[...], preferred_element_type=jnp.float32,
                  precision=lax.Precision.HIGHEST)
        + b_ref[...]
    )
    hm = jnp.where(mask_ref[...] != 0, out, gath_ref[...])
    hm_ref[...] = hm
    hr_ref[...] = jnp.maximum(hm, 0.0)


def _layer_call(h, agg, degp, ws, wn, b, mask, gath):
    return pl.pallas_call(
        _layer_body,
        grid=(N2 // RB,),
        in_specs=[
            pl.BlockSpec((RB, D), lambda i: (i, 0)),
            pl.BlockSpec((RB, D), lambda i: (i, 0)),
            pl.BlockSpec((RB, D), lambda i: (i, 0)),
            pl.BlockSpec((D, D), lambda i: (0, 0)),
            pl.BlockSpec((D, D), lambda i: (0, 0)),
            pl.BlockSpec((1, D), lambda i: (0, 0)),
            pl.BlockSpec((RB, 1), lambda i: (i, 0)),
            pl.BlockSpec((RB, D), lambda i: (i, 0)),
        ],
        out_specs=[
            pl.BlockSpec((RB, D), lambda i: (i, 0)),
            pl.BlockSpec((RB, D), lambda i: (i, 0)),
        ],
        out_shape=[
            jax.ShapeDtypeStruct((N2, D), jnp.float32),
            jax.ShapeDtypeStruct((N2, D), jnp.float32),
        ],
    )(h, agg, degp, ws, wn, b, mask, gath)


def _layer3_body(h_ref, agg_ref, degp_ref, ws_ref, wn_ref, b_ref, out_ref):
    deg = degp_ref[:, 0:1]
    hn = agg_ref[...] / jnp.maximum(deg, 1.0)
    out_ref[...] = (
        jnp.dot(h_ref[...], ws_ref[...], preferred_element_type=jnp.float32,
                precision=lax.Precision.HIGHEST)
        + jnp.dot(hn, wn_ref[...], preferred_element_type=jnp.float32,
                  precision=lax.Precision.HIGHEST)
        + b_ref[...]
    )


def _layer3_call(h, agg, degp, ws, wn, b):
    return pl.pallas_call(
        _layer3_body,
        grid=(N2 // RB,),
        in_specs=[
            pl.BlockSpec((RB, D), lambda i: (i, 0)),
            pl.BlockSpec((RB, D), lambda i: (i, 0)),
            pl.BlockSpec((RB, D), lambda i: (i, 0)),
            pl.BlockSpec((D, D), lambda i: (0, 0)),
            pl.BlockSpec((D, D), lambda i: (0, 0)),
            pl.BlockSpec((1, D), lambda i: (0, 0)),
        ],
        out_specs=[pl.BlockSpec((RB, D), lambda i: (i, 0))],
        out_shape=[jax.ShapeDtypeStruct((N2, D), jnp.float32)],
    )(h, agg, degp, ws, wn, b)[0]


CB = 2000


def _copy_body(a_ref, b_ref, oa_ref, ob_ref):
    oa_ref[...] = a_ref[...]
    ob_ref[...] = b_ref[...]


def _copy_tables(t1, t2):
    return pl.pallas_call(
        _copy_body,
        grid=(N_GLOBAL // CB,),
        in_specs=[
            pl.BlockSpec((CB, D), lambda i: (i, 0)),
            pl.BlockSpec((CB, D), lambda i: (i, 0)),
        ],
        out_specs=[
            pl.BlockSpec((CB, D), lambda i: (i, 0)),
            pl.BlockSpec((CB, D), lambda i: (i, 0)),
        ],
        out_shape=[
            jax.ShapeDtypeStruct((N_GLOBAL, D), jnp.float32),
            jax.ShapeDtypeStruct((N_GLOBAL, D), jnp.float32),
        ],
    )(t1, t2)


def _scatter_body(ids_ref, v1_ref, v2_ref, t1_ref, t2_ref, o1_ref, o2_ref):
    del ids_ref, t1_ref, t2_ref
    o1_ref[...] = v1_ref[...]
    o2_ref[...] = v2_ref[...]


def _scatter_tables(ids, v1, v2, t1, t2):
    grid_spec = pltpu.PrefetchScalarGridSpec(
        num_scalar_prefetch=1,
        grid=(N2,),
        in_specs=[
            pl.BlockSpec((1, 1, D), lambda i, ids: (i, 0, 0)),
            pl.BlockSpec((1, 1, D), lambda i, ids: (i, 0, 0)),
            pl.BlockSpec(memory_space=pl.ANY),
            pl.BlockSpec(memory_space=pl.ANY),
        ],
        out_specs=[
            pl.BlockSpec((1, 1, D), lambda i, ids: (ids[i], 0, 0)),
            pl.BlockSpec((1, 1, D), lambda i, ids: (ids[i], 0, 0)),
        ],
    )
    o1, o2 = pl.pallas_call(
        _scatter_body,
        grid_spec=grid_spec,
        out_shape=[
            jax.ShapeDtypeStruct((N_GLOBAL, 1, D), jnp.float32),
            jax.ShapeDtypeStruct((N_GLOBAL, 1, D), jnp.float32),
        ],
        input_output_aliases={3: 0, 4: 1},
    )(ids, v1.reshape(N2, 1, D), v2.reshape(N2, 1, D),
      t1.reshape(N_GLOBAL, 1, D), t2.reshape(N_GLOBAL, 1, D))
    return o1.reshape(N_GLOBAL, D), o2.reshape(N_GLOBAL, D)


def kernel(g2_feat, edge_index, g1_ids, g2_ids, gh_feat, gh2_feat,
           W1_self, W1_neigh, b1, W2_self, W2_neigh, b2,
           W3_self, W3_neigh, b3):
    src = edge_index[0]
    dst = edge_index[1]

    pad = E_PAD - E
    srcm = jnp.concatenate([src, jnp.zeros((pad,), jnp.int32)]).reshape(-1, B)
    dstm = jnp.concatenate([dst, jnp.full((pad,), N2P, jnp.int32)]).reshape(-1, B)

    memb = jnp.zeros((N_GLOBAL,), jnp.int32).at[g1_ids].set(1)
    mask = memb[g2_ids].astype(jnp.float32)[:, None]

    b1r = b1.reshape(1, D)
    b2r = b2.reshape(1, D)
    w3s = jnp.pad(W3_self, ((0, 0), (0, D - W3_self.shape[1])))
    w3n = jnp.pad(W3_neigh, ((0, 0), (0, D - W3_neigh.shape[1])))
    b3r = jnp.pad(b3, (0, D - b3.shape[0])).reshape(1, D)

    g2pad = jnp.concatenate(
        [g2_ids, jnp.zeros((N2P - N2,), jnp.int32)]).reshape(-1, B)
    degp, gath1, gath2 = _prep_call(dstm, g2pad, gh_feat, gh2_feat)
    gath1 = gath1[:N2]
    gath2 = gath2[:N2]

    agg1 = _agg_call(g2_feat, srcm, dstm)[:N2]
    h1m, h1r = _layer_call(g2_feat, agg1, degp, W1_self, W1_neigh, b1r,
                           mask, gath1)

    agg2 = _agg_call(h1r, srcm, dstm)[:N2]
    h2m, h2r = _layer_call(h1r, agg2, degp, W2_self, W2_neigh, b2r,
                           mask, gath2)

    agg3 = _agg_call(h2r, srcm, dstm)[:N2]
    h3 = _layer3_call(h2r, agg3, degp, w3s, w3n, b3r)[:, :W3_self.shape[1]]

    t1, t2 = _copy_tables(gh_feat, gh2_feat)
    gh_new, gh2_new = _scatter_tables(g2_ids, h1m, h2m, t1, t2)

    return h3, gh_new, gh2_new
